# Initial kernel scaffold; baseline (speedup 1.0000x reference)
import jax
import jax.numpy as jnp
from jax.experimental import pallas as pl


def kernel(rpn_bbox_pred, objectness, anchors, im_height, im_width):
    raise NotImplementedError("write your pallas kernel here")


# speedup vs baseline: 80.7610x; 80.7610x over previous
import functools

import jax
import jax.numpy as jnp
from jax import lax
from jax.experimental import pallas as pl
from jax.experimental.pallas import tpu as pltpu

_B = 8
_N = 20000
_PRE = 12000
_POST = 2000
_K = 512
_PAD = 12288
_NB = _PAD // _K
_NEG = -1e30
_TH = 0.7
_MINSZ = 16.0


def _decode_body(dx, dy, dw, dh, obj, a0, a1, a2, a3, hw,
                 x1o, y1o, x2o, y2o, so):
    h = hw[0, 0]
    w = hw[0, 1]
    widths = a2[...] - a0[...] + 1.0
    heights = a3[...] - a1[...] + 1.0
    ctr_x = a0[...] + 0.5 * widths
    ctr_y = a1[...] + 0.5 * heights
    pred_ctr_x = dx[...] * widths + ctr_x
    pred_ctr_y = dy[...] * heights + ctr_y
    pred_w = jnp.exp(dw[...]) * widths
    pred_h = jnp.exp(dh[...]) * heights
    x1 = jnp.clip(pred_ctr_x - 0.5 * pred_w, 0.0, w - 1.0)
    y1 = jnp.clip(pred_ctr_y - 0.5 * pred_h, 0.0, h - 1.0)
    x2 = jnp.clip(pred_ctr_x + 0.5 * pred_w, 0.0, w - 1.0)
    y2 = jnp.clip(pred_ctr_y + 0.5 * pred_h, 0.0, h - 1.0)
    keep = ((x2 - x1) >= _MINSZ) & ((y2 - y1) >= _MINSZ)
    x1o[...] = x1
    y1o[...] = y1
    x2o[...] = x2
    y2o[...] = y2
    so[...] = jnp.where(keep, obj[...], _NEG)


def _decode(rpn_bbox_pred, objectness, anchors, hw):
    f32 = jnp.float32
    outs = [jax.ShapeDtypeStruct((_B, _N), f32) for _ in range(5)]
    dx = rpn_bbox_pred[:, :, 0]
    dy = rpn_bbox_pred[:, :, 1]
    dw = rpn_bbox_pred[:, :, 2]
    dh = rpn_bbox_pred[:, :, 3]
    a0 = anchors[None, :, 0]
    a1 = anchors[None, :, 1]
    a2 = anchors[None, :, 2]
    a3 = anchors[None, :, 3]
    return pl.pallas_call(
        _decode_body,
        out_shape=tuple(outs),
    )(dx, dy, dw, dh, objectness, a0, a1, a2, a3, hw)


def _iou_tile(cx1, cy1, cx2, cy2, carea, rx1, ry1, rx2, ry2, rarea):
    xx1 = jnp.maximum(cx1, rx1)
    yy1 = jnp.maximum(cy1, ry1)
    xx2 = jnp.minimum(cx2, rx2)
    yy2 = jnp.minimum(cy2, ry2)
    inter = jnp.clip(xx2 - xx1, 0.0) * jnp.clip(yy2 - yy1, 0.0)
    iou = inter / (carea + rarea - inter + 1e-9)
    return (iou > _TH).astype(jnp.float32)


def _nms_body(x1r, y1r, x2r, y2r, sr, kept_ref):
    K = _K
    upper = (lax.broadcasted_iota(jnp.int32, (K, K), 0) <
             lax.broadcasted_iota(jnp.int32, (K, K), 1)).astype(jnp.float32)
    lower = (lax.broadcasted_iota(jnp.int32, (K, K), 0) >
             lax.broadcasted_iota(jnp.int32, (K, K), 1)).astype(jnp.float32)

    kept_ref[...] = jnp.zeros((1, 1, _PAD), jnp.float32)

    def transpose_rows(rows):
        return jnp.transpose(rows)

    def transpose_col(col):
        return jnp.transpose(col)

    def block_step(state):
        b, count = state
        i0 = b * K
        bx1 = x1r[0, 0:1, pl.ds(i0, K)]
        by1 = y1r[0, 0:1, pl.ds(i0, K)]
        bx2 = x2r[0, 0:1, pl.ds(i0, K)]
        by2 = y2r[0, 0:1, pl.ds(i0, K)]
        bs = sr[0, 0:1, pl.ds(i0, K)]
        cols = transpose_rows(
            jnp.concatenate([bx1, by1, bx2, by2, bs], axis=0))
        cx1 = cols[:, 0:1]
        cy1 = cols[:, 1:2]
        cx2 = cols[:, 2:3]
        cy2 = cols[:, 3:4]
        cs = cols[:, 4:5]
        carea = (cx2 - cx1) * (cy2 - cy1)
        barea = (bx2 - bx1) * (by2 - by1)
        valid_row = (bs > (_NEG * 0.5)).astype(jnp.float32)
        valid_col = (cs > (_NEG * 0.5)).astype(jnp.float32)

        def prev_step(p, act_col):
            j0 = p * K
            px1 = x1r[0, 0:1, pl.ds(j0, K)]
            py1 = y1r[0, 0:1, pl.ds(j0, K)]
            px2 = x2r[0, 0:1, pl.ds(j0, K)]
            py2 = y2r[0, 0:1, pl.ds(j0, K)]
            parea = (px2 - px1) * (py2 - py1)
            pk = kept_ref[0, 0:1, pl.ds(j0, K)]
            s = _iou_tile(cx1, cy1, cx2, cy2, carea,
                          px1, py1, px2, py2, parea)
            supp = jnp.max(s * pk, axis=1, keepdims=True)
            return act_col * (1.0 - supp)

        act_col = lax.fori_loop(0, b, prev_step, valid_col)
        act_row = transpose_col(act_col)

        s_sym = _iou_tile(cx1, cy1, cx2, cy2, carea,
                          bx1, by1, bx2, by2, barea)
        s_up = s_sym * upper
        s_lo = s_sym * lower

        def fix_cond(st):
            _, _, changed, it = st
            return changed & (it < K + 8)

        def fix_body(st):
            k_row, k_col, _, it = st
            supp_row = jnp.max(s_up * k_col, axis=0, keepdims=True)
            supp_col = jnp.max(s_lo * k_row, axis=1, keepdims=True)
            k_row_n = act_row * (1.0 - supp_row)
            k_col_n = act_col * (1.0 - supp_col)
            changed = jnp.any(k_row_n != k_row)
            return k_row_n, k_col_n, changed, it + 1

        k_row, k_col, _, _ = lax.while_loop(
            fix_cond, fix_body,
            (act_row, act_col, jnp.bool_(True), jnp.int32(0)))
        kept_ref[0, 0:1, pl.ds(i0, K)] = k_row
        count = count + jnp.sum(k_row)
        return b + 1, count

    def block_cond(state):
        b, count = state
        return (b < _NB) & (count < jnp.float32(_POST))

    lax.while_loop(block_cond, block_step, (jnp.int32(0), jnp.float32(0.0)))


def _nms(x1s, y1s, x2s, y2s, ss):
    spec = pl.BlockSpec((1, 1, _PAD), lambda b: (b, 0, 0))
    r = lambda v: v.reshape(_B, 1, _PAD)
    out = pl.pallas_call(
        _nms_body,
        grid=(_B,),
        in_specs=[spec] * 5,
        out_specs=spec,
        out_shape=jax.ShapeDtypeStruct((_B, 1, _PAD), jnp.float32),
    )(r(x1s), r(y1s), r(x2s), r(y2s), r(ss))
    return out.reshape(_B, _PAD)


def kernel(rpn_bbox_pred, objectness, anchors, im_height, im_width):
    f32 = jnp.float32
    hw = jnp.stack([jnp.asarray(im_height, f32),
                    jnp.asarray(im_width, f32)]).reshape(1, 2)
    x1, y1, x2, y2, scores = _decode(rpn_bbox_pred, objectness, anchors, hw)

    _, order = lax.top_k(scores, _PRE)
    ssort = jnp.take_along_axis(scores, order, axis=1)
    pad = ((0, 0), (0, _PAD - _PRE))
    ssort = jnp.pad(ssort, pad, constant_values=_NEG)
    planes = []
    for v in (x1, y1, x2, y2):
        vs = jnp.take_along_axis(v, order, axis=1)
        planes.append(jnp.pad(vs, pad))
    x1s, y1s, x2s, y2s = planes

    kept = _nms(x1s, y1s, x2s, y2s, ssort)

    keptb = kept > 0.5
    pos = jnp.cumsum(keptb.astype(jnp.int32), axis=1) - 1
    dest = jnp.where(keptb & (pos < _POST), pos, _POST + 1)
    boxes4 = jnp.stack([x1s, y1s, x2s, y2s], axis=-1)
    bidx = jnp.arange(_B, dtype=jnp.int32)[:, None]
    out = jnp.zeros((_B, _POST, 4), f32)
    out = out.at[bidx, dest].set(boxes4, mode="drop")
    return out


# speedup vs baseline: 123.2403x; 1.5260x over previous
import functools

import jax
import jax.numpy as jnp
from jax import lax
from jax.experimental import pallas as pl
from jax.experimental.pallas import tpu as pltpu
from jax.experimental.pallas import tpu_sc as plsc

_B = 8
_N = 20000
_PRE = 12000
_POST = 2000
_K = 512
_PAD = 12288
_NB = _PAD // _K
_NEG = -1e30
_TH = 0.7
_MINSZ = 16.0


def _decode_body(dx, dy, dw, dh, obj, a0, a1, a2, a3, hw,
                 x1o, y1o, x2o, y2o, so):
    h = hw[0, 0]
    w = hw[0, 1]
    widths = a2[...] - a0[...] + 1.0
    heights = a3[...] - a1[...] + 1.0
    ctr_x = a0[...] + 0.5 * widths
    ctr_y = a1[...] + 0.5 * heights
    pred_ctr_x = dx[...] * widths + ctr_x
    pred_ctr_y = dy[...] * heights + ctr_y
    pred_w = jnp.exp(dw[...]) * widths
    pred_h = jnp.exp(dh[...]) * heights
    x1 = jnp.clip(pred_ctr_x - 0.5 * pred_w, 0.0, w - 1.0)
    y1 = jnp.clip(pred_ctr_y - 0.5 * pred_h, 0.0, h - 1.0)
    x2 = jnp.clip(pred_ctr_x + 0.5 * pred_w, 0.0, w - 1.0)
    y2 = jnp.clip(pred_ctr_y + 0.5 * pred_h, 0.0, h - 1.0)
    keep = ((x2 - x1) >= _MINSZ) & ((y2 - y1) >= _MINSZ)
    x1o[...] = x1
    y1o[...] = y1
    x2o[...] = x2
    y2o[...] = y2
    so[...] = jnp.where(keep, obj[...], _NEG)


def _decode(rpn_bbox_pred, objectness, anchors, hw):
    f32 = jnp.float32
    outs = [jax.ShapeDtypeStruct((_B, _N), f32) for _ in range(5)]
    dx = rpn_bbox_pred[:, :, 0]
    dy = rpn_bbox_pred[:, :, 1]
    dw = rpn_bbox_pred[:, :, 2]
    dh = rpn_bbox_pred[:, :, 3]
    a0 = anchors[None, :, 0]
    a1 = anchors[None, :, 1]
    a2 = anchors[None, :, 2]
    a3 = anchors[None, :, 3]
    return pl.pallas_call(
        _decode_body,
        out_shape=tuple(outs),
    )(dx, dy, dw, dh, objectness, a0, a1, a2, a3, hw)


def _iou_tile(cx1, cy1, cx2, cy2, carea, rx1, ry1, rx2, ry2, rarea):
    xx1 = jnp.maximum(cx1, rx1)
    yy1 = jnp.maximum(cy1, ry1)
    xx2 = jnp.minimum(cx2, rx2)
    yy2 = jnp.minimum(cy2, ry2)
    inter = jnp.clip(xx2 - xx1, 0.0) * jnp.clip(yy2 - yy1, 0.0)
    iou = inter / (carea + rarea - inter + 1e-9)
    return (iou > _TH).astype(jnp.float32)


def _nms_body(x1r, y1r, x2r, y2r, sr, kept_ref):
    K = _K
    upper = (lax.broadcasted_iota(jnp.int32, (K, K), 0) <
             lax.broadcasted_iota(jnp.int32, (K, K), 1)).astype(jnp.float32)
    lower = (lax.broadcasted_iota(jnp.int32, (K, K), 0) >
             lax.broadcasted_iota(jnp.int32, (K, K), 1)).astype(jnp.float32)

    kept_ref[...] = jnp.zeros((1, 1, _PAD), jnp.float32)

    def transpose_rows(rows):
        return jnp.transpose(rows)

    def transpose_col(col):
        return jnp.transpose(col)

    def block_step(state):
        b, count = state
        i0 = b * K
        bx1 = x1r[0, 0:1, pl.ds(i0, K)]
        by1 = y1r[0, 0:1, pl.ds(i0, K)]
        bx2 = x2r[0, 0:1, pl.ds(i0, K)]
        by2 = y2r[0, 0:1, pl.ds(i0, K)]
        bs = sr[0, 0:1, pl.ds(i0, K)]
        cols = transpose_rows(
            jnp.concatenate([bx1, by1, bx2, by2, bs], axis=0))
        cx1 = cols[:, 0:1]
        cy1 = cols[:, 1:2]
        cx2 = cols[:, 2:3]
        cy2 = cols[:, 3:4]
        cs = cols[:, 4:5]
        carea = (cx2 - cx1) * (cy2 - cy1)
        barea = (bx2 - bx1) * (by2 - by1)
        valid_row = (bs > (_NEG * 0.5)).astype(jnp.float32)
        valid_col = (cs > (_NEG * 0.5)).astype(jnp.float32)

        def prev_step(p, act_col):
            j0 = p * K
            px1 = x1r[0, 0:1, pl.ds(j0, K)]
            py1 = y1r[0, 0:1, pl.ds(j0, K)]
            px2 = x2r[0, 0:1, pl.ds(j0, K)]
            py2 = y2r[0, 0:1, pl.ds(j0, K)]
            parea = (px2 - px1) * (py2 - py1)
            pk = kept_ref[0, 0:1, pl.ds(j0, K)]
            s = _iou_tile(cx1, cy1, cx2, cy2, carea,
                          px1, py1, px2, py2, parea)
            supp = jnp.max(s * pk, axis=1, keepdims=True)
            return act_col * (1.0 - supp)

        act_col = lax.fori_loop(0, b, prev_step, valid_col)
        act_row = transpose_col(act_col)

        s_sym = _iou_tile(cx1, cy1, cx2, cy2, carea,
                          bx1, by1, bx2, by2, barea)
        s_up = s_sym * upper
        s_lo = s_sym * lower

        def fix_cond(st):
            _, _, changed, it = st
            return changed & (it < K + 8)

        def fix_body(st):
            k_row, k_col, _, it = st
            supp_row = jnp.max(s_up * k_col, axis=0, keepdims=True)
            supp_col = jnp.max(s_lo * k_row, axis=1, keepdims=True)
            k_row_n = act_row * (1.0 - supp_row)
            k_col_n = act_col * (1.0 - supp_col)
            changed = jnp.any(k_row_n != k_row)
            return k_row_n, k_col_n, changed, it + 1

        k_row, k_col, _, _ = lax.while_loop(
            fix_cond, fix_body,
            (act_row, act_col, jnp.bool_(True), jnp.int32(0)))
        kept_ref[0, 0:1, pl.ds(i0, K)] = k_row
        count = count + jnp.sum(k_row)
        return b + 1, count

    def block_cond(state):
        b, count = state
        return (b < _NB) & (count < jnp.float32(_POST))

    lax.while_loop(block_cond, block_step, (jnp.int32(0), jnp.float32(0.0)))


def _nms(x1s, y1s, x2s, y2s, ss):
    spec = pl.BlockSpec((1, 1, _PAD), lambda b: (b, 0, 0))
    r = lambda v: v.reshape(_B, 1, _PAD)
    out = pl.pallas_call(
        _nms_body,
        grid=(_B,),
        in_specs=[spec] * 5,
        out_specs=spec,
        out_shape=jax.ShapeDtypeStruct((_B, 1, _PAD), jnp.float32),
    )(r(x1s), r(y1s), r(x2s), r(y2s), r(ss))
    return out.reshape(_B, _PAD)


_SENT = 2047


def _compact_sc(dest, x1s, y1s, x2s, y2s):
    i32 = jnp.int32
    f32 = jnp.float32
    mesh = plsc.VectorSubcoreMesh(core_axis_name="c", subcore_axis_name="s")
    zeros = jnp.zeros((2048 * 4,), f32)

    @functools.partial(
        pl.kernel, mesh=mesh,
        compiler_params=pltpu.CompilerParams(use_tc_tiling_on_sc=False,
                                             needs_layout_passes=False),
        out_type=jax.ShapeDtypeStruct((_B, _POST * 4), f32),
        scratch_types=[
            pltpu.VMEM((_PAD,), i32),
            pltpu.VMEM((_PAD,), f32),
            pltpu.VMEM((_PAD,), f32),
            pltpu.VMEM((_PAD,), f32),
            pltpu.VMEM((_PAD,), f32),
            pltpu.VMEM((2048 * 4,), f32),
        ],
    )
    def body(dest_hbm, x1h, y1h, x2h, y2h, z_hbm, out_hbm,
             dv, v0, v1, v2, v3, outbuf):
        wid = lax.axis_index("s") * 2 + lax.axis_index("c")

        @pl.when(wid < _B)
        def _():
            pltpu.sync_copy(dest_hbm.at[wid], dv)
            pltpu.sync_copy(x1h.at[wid], v0)
            pltpu.sync_copy(y1h.at[wid], v1)
            pltpu.sync_copy(x2h.at[wid], v2)
            pltpu.sync_copy(y2h.at[wid], v3)
            pltpu.sync_copy(z_hbm, outbuf)

            def step(t, carry):
                base = t * 16
                d16 = dv[pl.ds(base, 16)]
                fx = d16 * 4
                plsc.store_scatter(outbuf, [fx], v0[pl.ds(base, 16)])
                plsc.store_scatter(outbuf, [fx + 1], v1[pl.ds(base, 16)])
                plsc.store_scatter(outbuf, [fx + 2], v2[pl.ds(base, 16)])
                plsc.store_scatter(outbuf, [fx + 3], v3[pl.ds(base, 16)])
                return carry

            lax.fori_loop(0, _PAD // 16, step, 0)
            pltpu.sync_copy(outbuf.at[pl.ds(0, _POST * 4)], out_hbm.at[wid])

    return body(dest, x1s, y1s, x2s, y2s, zeros)


def kernel(rpn_bbox_pred, objectness, anchors, im_height, im_width):
    f32 = jnp.float32
    hw = jnp.stack([jnp.asarray(im_height, f32),
                    jnp.asarray(im_width, f32)]).reshape(1, 2)
    x1, y1, x2, y2, scores = _decode(rpn_bbox_pred, objectness, anchors, hw)

    _, order = lax.top_k(scores, _PRE)
    ssort = jnp.take_along_axis(scores, order, axis=1)
    pad = ((0, 0), (0, _PAD - _PRE))
    ssort = jnp.pad(ssort, pad, constant_values=_NEG)
    planes = []
    for v in (x1, y1, x2, y2):
        vs = jnp.take_along_axis(v, order, axis=1)
        planes.append(jnp.pad(vs, pad))
    x1s, y1s, x2s, y2s = planes

    kept = _nms(x1s, y1s, x2s, y2s, ssort)

    keptb = kept > 0.5
    pos = jnp.cumsum(keptb.astype(jnp.int32), axis=1) - 1
    dest = jnp.where(keptb & (pos < _POST), pos, _SENT)
    out = _compact_sc(dest, x1s, y1s, x2s, y2s)
    return out.reshape(_B, _POST, 4)
